# baseline (speedup 1.0000x reference)
import functools
import jax
import jax.numpy as jnp
from jax import lax
from jax.experimental import pallas as pl
from jax.experimental.pallas import tpu as pltpu
from jax.experimental.pallas import tpu_sc as plsc

N_NODES = 10000
D = 128
N_EDGES = 320000
NC = 2
NS = 16
NW = NC * NS
EPW = N_EDGES // NW
K = 64
NCHUNK = EPW // K
KT = EPW - NCHUNK * K
RING = 5
NMAIN = (NCHUNK - 1) // RING * RING
ZR = 624
ZT = N_NODES - NS * ZR
ZB = 24


def _sc_segsum(x, eidx):
  mesh = plsc.VectorSubcoreMesh(core_axis_name="c", subcore_axis_name="s")

  @functools.partial(
      pl.kernel,
      mesh=mesh,
      out_type=[
          jax.ShapeDtypeStruct((NC, N_NODES, D), jnp.float32),
          jax.ShapeDtypeStruct((NC * N_NODES,), jnp.float32),
      ],
      scratch_types=[
          pltpu.VMEM((RING, K), jnp.int32),
          pltpu.VMEM((RING, K), jnp.int32),
          pltpu.VMEM((1, KT), jnp.int32),
          [pltpu.VMEM((K, D), jnp.float32) for _ in range(RING)],
          pltpu.VMEM((K,), jnp.float32),
          pltpu.VMEM((ZR,), jnp.float32),
          pltpu.VMEM((ZB, D), jnp.float32),
          pltpu.VMEM_SHARED((N_NODES, D), jnp.float32),
          pltpu.VMEM_SHARED((N_NODES,), jnp.float32),
          [pltpu.SemaphoreType.DMA for _ in range(RING)],
          [pltpu.SemaphoreType.DMA for _ in range(RING)],
          [pltpu.SemaphoreType.DMA for _ in range(RING)],
      ],
  )
  def k(x_hbm, eidx_hbm, part_hbm, degw_hbm,
        colu, rowu, rowt, bufs, onesv, dzb, zb2, acc, dega,
        gsems, rsems, csems):
    c = lax.axis_index("c")
    s = lax.axis_index("s")
    wid = c * NS + s
    base = pl.multiple_of(wid * EPW, 8)

    tbase = pl.multiple_of(wid * EPW + NCHUNK * K, 8)
    pltpu.sync_copy(eidx_hbm.at[pl.ds(tbase, KT)], rowt.at[0])

    for j in range(RING):
      off = pl.multiple_of(wid * EPW + j * K, 8)
      pltpu.async_copy(eidx_hbm.at[pl.ds(N_EDGES + off, K)], colu.at[j],
                       csems[j])
      pltpu.async_copy(eidx_hbm.at[pl.ds(off, K)], rowu.at[j], rsems[j])
    for j in range(RING - 1):
      pltpu.make_async_copy(eidx_hbm.at[pl.ds(base, K)], colu.at[j],
                            csems[j]).wait()
      pltpu.async_copy(x_hbm.at[colu.at[j]], bufs[j], gsems[j])

    def zrow(i, carry):
      for j in range(D // 16):
        zb2[i, pl.ds(j * 16, 16)] = jnp.zeros((16,), jnp.float32)
      return carry
    lax.fori_loop(0, ZB, zrow, 0)
    def zdeg(i, carry):
      dzb[pl.ds(i * 16, 16)] = jnp.zeros((16,), jnp.float32)
      return carry
    lax.fori_loop(0, ZR // 16, zdeg, 0)

    for jj in range(ZR // ZB):
      pltpu.sync_copy(zb2, acc.at[pl.ds(s * ZR + jj * ZB, ZB)])
    pltpu.sync_copy(dzb, dega.at[pl.ds(s * ZR, ZR)])
    @pl.when(s == 0)
    def _():
      pltpu.sync_copy(zb2.at[pl.ds(0, ZT)], acc.at[pl.ds(NS * ZR, ZT)])
      pltpu.sync_copy(dzb.at[pl.ds(0, ZT)], dega.at[pl.ds(NS * ZR, ZT)])

    for j in range(K // 16):
      onesv[pl.ds(j * 16, 16)] = jnp.ones((16,), jnp.float32)
    plsc.subcore_barrier()

    def outer(o, carry):
      for j in range(RING):
        i = o * RING + j
        jn = (j + RING - 1) % RING
        @pl.when(i + RING - 1 < NCHUNK)
        def _():
          pltpu.make_async_copy(eidx_hbm.at[pl.ds(base, K)], colu.at[jn],
                                csems[jn]).wait()
          pltpu.async_copy(x_hbm.at[colu.at[jn]], bufs[jn], gsems[jn])
        pltpu.make_async_copy(x_hbm.at[colu.at[j]], bufs[j],
                              gsems[j]).wait()
        pltpu.make_async_copy(eidx_hbm.at[pl.ds(base, K)], rowu.at[j],
                              rsems[j]).wait()
        pltpu.sync_copy(bufs[j], acc.at[rowu.at[j]], add=True)
        pltpu.sync_copy(onesv, dega.at[rowu.at[j]], add=True)
        @pl.when(i + RING < NCHUNK)
        def _():
          noff = pl.multiple_of(wid * EPW + (i + RING) * K, 8)
          pltpu.async_copy(eidx_hbm.at[pl.ds(N_EDGES + noff, K)],
                           colu.at[j], csems[j])
          pltpu.async_copy(eidx_hbm.at[pl.ds(noff, K)], rowu.at[j],
                           rsems[j])
      return carry
    lax.fori_loop(0, NMAIN // RING, outer, 0)

    pltpu.make_async_copy(x_hbm.at[colu.at[0]], bufs[0], gsems[0]).wait()
    pltpu.make_async_copy(eidx_hbm.at[pl.ds(base, K)], rowu.at[0],
                          rsems[0]).wait()
    pltpu.sync_copy(bufs[0], acc.at[rowu.at[0]], add=True)
    pltpu.sync_copy(onesv, dega.at[rowu.at[0]], add=True)

    pltpu.sync_copy(eidx_hbm.at[pl.ds(N_EDGES + tbase, KT)],
                    colu.at[0, pl.ds(0, KT)])
    pltpu.async_copy(x_hbm.at[colu.at[0, pl.ds(0, KT)]],
                     bufs[0].at[pl.ds(0, KT)], gsems[0])
    pltpu.make_async_copy(x_hbm.at[colu.at[0, pl.ds(0, KT)]],
                          bufs[0].at[pl.ds(0, KT)], gsems[0]).wait()
    pltpu.sync_copy(bufs[0].at[pl.ds(0, KT)], acc.at[rowt.at[0]], add=True)
    pltpu.sync_copy(onesv.at[pl.ds(0, KT)], dega.at[rowt.at[0]], add=True)

    plsc.subcore_barrier()

    pltpu.sync_copy(acc.at[pl.ds(s * ZR, ZR)],
                    part_hbm.at[c, pl.ds(s * ZR, ZR)])
    pltpu.sync_copy(dega.at[pl.ds(s * ZR, ZR)], dzb)
    doff = pl.multiple_of(c * N_NODES + s * ZR, 8)
    pltpu.sync_copy(dzb, degw_hbm.at[pl.ds(doff, ZR)])
    @pl.when(s == 0)
    def _():
      pltpu.sync_copy(acc.at[pl.ds(NS * ZR, ZT)],
                      part_hbm.at[c, pl.ds(NS * ZR, ZT)])
      pltpu.sync_copy(dega.at[pl.ds(NS * ZR, ZT)], dzb.at[pl.ds(0, ZT)])
      toff = pl.multiple_of(c * N_NODES + NS * ZR, 8)
      pltpu.sync_copy(dzb.at[pl.ds(0, ZT)], degw_hbm.at[pl.ds(toff, ZT)])

  return k(x, eidx)


def _tc_combine(part, degw, x, W):
  NB = 2000

  def body(part_ref, degw_ref, x_ref, w_ref, o_ref):
    p = part_ref[0] + part_ref[1]
    d = degw_ref[0] + degw_ref[1]
    agg = p - d * x_ref[...]
    o_ref[...] = lax.dot_general(
        agg, w_ref[...], (((1,), (1,)), ((), ())),
        preferred_element_type=jnp.float32)

  return pl.pallas_call(
      body,
      grid=(N_NODES // NB,),
      in_specs=[
          pl.BlockSpec((NC, NB, D), lambda i: (0, i, 0)),
          pl.BlockSpec((NC, NB, 1), lambda i: (0, i, 0)),
          pl.BlockSpec((NB, D), lambda i: (i, 0)),
          pl.BlockSpec((D, D), lambda i: (0, 0)),
      ],
      out_specs=pl.BlockSpec((NB, D), lambda i: (i, 0)),
      out_shape=jax.ShapeDtypeStruct((N_NODES, D), jnp.float32),
  )(part, degw.reshape(NC, N_NODES, 1), x, W)


def kernel(x, edge_index, W, b):
  eidx = edge_index.astype(jnp.int32).reshape(2 * N_EDGES)
  part, degw = _sc_segsum(x, eidx)
  return _tc_combine(part, degw, x, W)


# speedup vs baseline: 1.4513x; 1.4513x over previous
import functools
import jax
import jax.numpy as jnp
from jax import lax
from jax.experimental import pallas as pl
from jax.experimental.pallas import tpu as pltpu
from jax.experimental.pallas import tpu_sc as plsc

N_NODES = 10000
D = 128
N_EDGES = 320000
NC = 2
NS = 16
NW = NC * NS
EPW = N_EDGES // NW
K = 64
NCHUNK = EPW // K
KT = EPW - NCHUNK * K
RING = 4
ZR = 624
ZT = N_NODES - NS * ZR
ZB = 48


def _sc_segsum(x, eidx):
  mesh = plsc.VectorSubcoreMesh(core_axis_name="c", subcore_axis_name="s")

  @functools.partial(
      pl.kernel,
      mesh=mesh,
      out_type=[
          jax.ShapeDtypeStruct((NC, N_NODES, D), jnp.float32),
          jax.ShapeDtypeStruct((NC * N_NODES,), jnp.float32),
      ],
      scratch_types=[
          pltpu.VMEM((EPW,), jnp.int32),
          pltpu.VMEM((RING, K), jnp.int32),
          pltpu.VMEM((1, KT), jnp.int32),
          [pltpu.VMEM((K, D), jnp.float32) for _ in range(RING)],
          pltpu.VMEM((K,), jnp.float32),
          pltpu.VMEM((ZR,), jnp.float32),
          pltpu.VMEM((ZB, D), jnp.float32),
          pltpu.VMEM_SHARED((N_NODES, D), jnp.float32),
          pltpu.VMEM_SHARED((N_NODES,), jnp.float32),
          [pltpu.SemaphoreType.DMA for _ in range(RING)],
          [pltpu.SemaphoreType.DMA for _ in range(RING)],
          [pltpu.SemaphoreType.DMA for _ in range(RING)],
      ],
  )
  def k(x_hbm, eidx_hbm, part_hbm, degw_hbm,
        coli, rowu, rowt, bufs, onesv, dzb, zb2, acc, dega, gsems, rsems,
        dsems):
    c = lax.axis_index("c")
    s = lax.axis_index("s")
    wid = c * NS + s
    base = pl.multiple_of(wid * EPW, 8)

    pltpu.sync_copy(eidx_hbm.at[pl.ds(N_EDGES + base, EPW)], coli)
    tbase = pl.multiple_of(wid * EPW + NCHUNK * K, 8)
    pltpu.sync_copy(eidx_hbm.at[pl.ds(tbase, KT)], rowt.at[0])

    for j in range(RING):
      off = pl.multiple_of(wid * EPW + j * K, 8)
      pltpu.async_copy(eidx_hbm.at[pl.ds(off, K)], rowu.at[j], rsems[j])
      pltpu.async_copy(x_hbm.at[coli.at[pl.ds(j * K, K)]], bufs[j], gsems[j])

    def zrow(i, carry):
      for j in range(D // 16):
        zb2[i, pl.ds(j * 16, 16)] = jnp.zeros((16,), jnp.float32)
      return carry
    lax.fori_loop(0, ZB, zrow, 0)
    def zdeg(i, carry):
      dzb[pl.ds(i * 16, 16)] = jnp.zeros((16,), jnp.float32)
      return carry
    lax.fori_loop(0, ZR // 16, zdeg, 0)

    for jj in range(ZR // ZB):
      pltpu.sync_copy(zb2, acc.at[pl.ds(s * ZR + jj * ZB, ZB)])
    pltpu.sync_copy(dzb, dega.at[pl.ds(s * ZR, ZR)])
    @pl.when(s == 0)
    def _():
      pltpu.sync_copy(zb2.at[pl.ds(0, ZT)], acc.at[pl.ds(NS * ZR, ZT)])
      pltpu.sync_copy(dzb.at[pl.ds(0, ZT)], dega.at[pl.ds(NS * ZR, ZT)])

    for j in range(K // 16):
      onesv[pl.ds(j * 16, 16)] = jnp.ones((16,), jnp.float32)
    plsc.subcore_barrier()

    def outer(o, carry):
      for j in range(RING):
        i = o * RING + j
        pltpu.make_async_copy(x_hbm.at[coli.at[pl.ds(0, K)]], bufs[j],
                              gsems[j]).wait()
        pltpu.make_async_copy(eidx_hbm.at[pl.ds(base, K)], rowu.at[j],
                              rsems[j]).wait()
        pltpu.async_copy(onesv, dega.at[rowu.at[j]], dsems[j], add=True)
        pltpu.sync_copy(bufs[j], acc.at[rowu.at[j]], add=True)
        pltpu.make_async_copy(onesv, dega.at[rowu.at[j]], dsems[j]).wait()
        nxt = i + RING
        @pl.when(nxt < NCHUNK)
        def _():
          noff = pl.multiple_of(wid * EPW + nxt * K, 8)
          pltpu.async_copy(eidx_hbm.at[pl.ds(noff, K)], rowu.at[j],
                           rsems[j])
          pltpu.async_copy(x_hbm.at[coli.at[pl.ds(nxt * K, K)]], bufs[j],
                           gsems[j])
      return carry
    lax.fori_loop(0, NCHUNK // RING, outer, 0)

    pltpu.async_copy(x_hbm.at[coli.at[pl.ds(NCHUNK * K, KT)]],
                     bufs[0].at[pl.ds(0, KT)], gsems[0])
    pltpu.make_async_copy(x_hbm.at[coli.at[pl.ds(0, KT)]],
                          bufs[0].at[pl.ds(0, KT)], gsems[0]).wait()
    pltpu.sync_copy(bufs[0].at[pl.ds(0, KT)], acc.at[rowt.at[0]], add=True)
    pltpu.sync_copy(onesv.at[pl.ds(0, KT)], dega.at[rowt.at[0]], add=True)

    plsc.subcore_barrier()

    pltpu.sync_copy(acc.at[pl.ds(s * ZR, ZR)],
                    part_hbm.at[c, pl.ds(s * ZR, ZR)])
    pltpu.sync_copy(dega.at[pl.ds(s * ZR, ZR)], dzb)
    doff = pl.multiple_of(c * N_NODES + s * ZR, 8)
    pltpu.sync_copy(dzb, degw_hbm.at[pl.ds(doff, ZR)])
    @pl.when(s == 0)
    def _():
      pltpu.sync_copy(acc.at[pl.ds(NS * ZR, ZT)],
                      part_hbm.at[c, pl.ds(NS * ZR, ZT)])
      pltpu.sync_copy(dega.at[pl.ds(NS * ZR, ZT)], dzb.at[pl.ds(0, ZT)])
      toff = pl.multiple_of(c * N_NODES + NS * ZR, 8)
      pltpu.sync_copy(dzb.at[pl.ds(0, ZT)], degw_hbm.at[pl.ds(toff, ZT)])

  return k(x, eidx)


def _tc_combine(part, degw, x, W):
  NB = 2000

  def body(part_ref, degw_ref, x_ref, w_ref, o_ref):
    p = part_ref[0] + part_ref[1]
    d = degw_ref[0] + degw_ref[1]
    agg = p - d * x_ref[...]
    o_ref[...] = lax.dot_general(
        agg, w_ref[...], (((1,), (1,)), ((), ())),
        preferred_element_type=jnp.float32)

  return pl.pallas_call(
      body,
      grid=(N_NODES // NB,),
      in_specs=[
          pl.BlockSpec((NC, NB, D), lambda i: (0, i, 0)),
          pl.BlockSpec((NC, NB, 1), lambda i: (0, i, 0)),
          pl.BlockSpec((NB, D), lambda i: (i, 0)),
          pl.BlockSpec((D, D), lambda i: (0, 0)),
      ],
      out_specs=pl.BlockSpec((NB, D), lambda i: (i, 0)),
      out_shape=jax.ShapeDtypeStruct((N_NODES, D), jnp.float32),
  )(part, degw.reshape(NC, N_NODES, 1), x, W)


def kernel(x, edge_index, W, b):
  eidx = edge_index.astype(jnp.int32).reshape(2 * N_EDGES)
  part, degw = _sc_segsum(x, eidx)
  return _tc_combine(part, degw, x, W)
